# Initial kernel scaffold; baseline (speedup 1.0000x reference)
import functools

import jax
import jax.numpy as jnp
from jax import lax
from jax.experimental import pallas as pl
from jax.experimental.pallas import tpu as pltpu
from jax.experimental.pallas import tpu_sc as plsc

N_NODES = 100000
G_SEG = 64
N_PAD = 100352
NR = 784
NPW = N_PAD // 16
NCORE = 2
NSUB = 16
NW = NCORE * NSUB
CHUNK = 128
INNER = 8


def _mesh():
    return plsc.VectorSubcoreMesh(
        core_axis_name="c", subcore_axis_name="s",
        num_cores=NCORE, num_subcores=NSUB)


def _zero_vmem_1d(ref, n):
    def body(i, _):
        ref[pl.ds(i * 16, 16)] = jnp.zeros((16,), jnp.float32)
        return 0
    lax.fori_loop(0, n // 16, body, 0)


def _sc_deg_body(rows_per_worker, dst_hbm, out_hbm, dstbuf, ones_v, zbuf,
                 obuf, deg_sp, gsem):
    c = lax.axis_index("c")
    s = lax.axis_index("s")
    w = c * NSUB + s

    _zero_vmem_1d(zbuf, NPW)

    def fill_ones(i, _):
        ones_v[pl.ds(i * 16, 16)] = jnp.ones((16,), jnp.float32)
        return 0
    lax.fori_loop(0, CHUNK // 16, fill_ones, 0)

    pltpu.sync_copy(zbuf, deg_sp.at[pl.ds(s * NPW, NPW)])
    plsc.subcore_barrier()

    n_outer = rows_per_worker // INNER
    row0 = w * rows_per_worker

    def outer(k, _):
        r0 = row0 + k * INNER
        pltpu.sync_copy(dst_hbm.at[pl.ds(r0, INNER)], dstbuf)
        for j in range(INNER):
            pltpu.sync_copy(ones_v, deg_sp.at[dstbuf.at[j]], add=True)
        return 0
    lax.fori_loop(0, n_outer, outer, 0)

    plsc.subcore_barrier()
    pltpu.sync_copy(deg_sp.at[pl.ds(s * NPW, NPW)], obuf)
    pltpu.sync_copy(obuf, out_hbm.at[c, pl.ds(s * NPW, NPW)])


def _sc_deg(dst2d):
    rows = dst2d.shape[0]
    rpw = rows // NW
    body = functools.partial(_sc_deg_body, rpw)
    return pl.kernel(
        body,
        out_type=jax.ShapeDtypeStruct((NCORE, N_PAD), jnp.float32),
        mesh=_mesh(),
        scratch_types=[
            pltpu.VMEM((INNER, CHUNK), jnp.int32),
            pltpu.VMEM((CHUNK,), jnp.float32),
            pltpu.VMEM((NPW,), jnp.float32),
            pltpu.VMEM((NPW,), jnp.float32),
            pltpu.VMEM_SHARED((N_PAD,), jnp.float32),
            pltpu.SemaphoreType.DMA,
        ],
    )(dst2d)


def _sc_spmv_body(rows_per_worker, src_hbm, dst_hbm, vals_hbm, zeros_hbm,
                  out_hbm, srcbuf, dstbuf, gbuf, zbuf, y_sp, gsem):
    c = lax.axis_index("c")
    s = lax.axis_index("s")
    w = c * NSUB + s

    pltpu.sync_copy(zeros_hbm.at[pl.ds(s * NPW, NPW), :], zbuf)
    pltpu.sync_copy(zbuf, y_sp.at[pl.ds(s * NPW, NPW), :])
    plsc.subcore_barrier()

    n_outer = rows_per_worker // INNER
    row0 = w * rows_per_worker

    def outer(k, _):
        r0 = row0 + k * INNER
        pltpu.sync_copy(src_hbm.at[pl.ds(r0, INNER)], srcbuf)
        pltpu.sync_copy(dst_hbm.at[pl.ds(r0, INNER)], dstbuf)
        descs = []
        for j in range(INNER):
            descs.append(pltpu.async_copy(
                vals_hbm.at[srcbuf.at[j]], gbuf.at[j], gsem))
        for d in descs:
            d.wait()
        for j in range(INNER):
            pltpu.sync_copy(gbuf.at[j], y_sp.at[dstbuf.at[j]], add=True)
        return 0
    lax.fori_loop(0, n_outer, outer, 0)

    plsc.subcore_barrier()
    pltpu.sync_copy(y_sp.at[pl.ds(s * NPW, NPW), :], zbuf)
    pltpu.sync_copy(zbuf, out_hbm.at[c, pl.ds(s * NPW, NPW), :])


def _sc_spmv(src2d, dst2d, vals, zeros2):
    rows = src2d.shape[0]
    rpw = rows // NW
    body = functools.partial(_sc_spmv_body, rpw)
    return pl.kernel(
        body,
        out_type=jax.ShapeDtypeStruct((NCORE, N_PAD, 2), jnp.float32),
        mesh=_mesh(),
        scratch_types=[
            pltpu.VMEM((INNER, CHUNK), jnp.int32),
            pltpu.VMEM((INNER, CHUNK), jnp.int32),
            pltpu.VMEM((INNER, CHUNK, 2), jnp.float32),
            pltpu.VMEM((NPW, 2), jnp.float32),
            pltpu.VMEM_SHARED((N_PAD, 2), jnp.float32),
            pltpu.SemaphoreType.DMA,
        ],
    )(src2d, dst2d, vals, zeros2)


def _tc_prep_body(d0, d1, x0, x1, dis_o, inv_o, xs0_o, xs1_o):
    deg = d0[...] + d1[...] + 1.0
    dis = lax.rsqrt(deg)
    inv = 1.0 / deg
    dis_o[...] = dis
    inv_o[...] = inv
    xs0_o[...] = x0[...] * dis
    xs1_o[...] = x1[...] * dis


def _tc_prep(d0, d1, x0, x1):
    sds = jax.ShapeDtypeStruct((NR, 128), jnp.float32)
    return pl.pallas_call(
        _tc_prep_body,
        out_shape=[sds, sds, sds, sds],
    )(d0, d1, x0, x1)


def _tc_mid_body(a00, a01, a10, a11, x0, x1, dis, inv, W1, b1, W2,
                 ts0_o, ts1_o, tf0_o, tf1_o):
    disv = dis[...]
    invv = inv[...]
    y0 = disv * (a00[...] + a10[...]) + x0[...] * invv
    y1 = disv * (a01[...] + a11[...]) + x1[...] * invv
    t0 = jnp.zeros_like(y0)
    t1 = jnp.zeros_like(y0)
    for j in range(64):
        h = jnp.maximum(y0 * W1[0, j] + y1 * W1[1, j] + b1[j], 0.0)
        t0 = t0 + h * W2[j, 0]
        t1 = t1 + h * W2[j, 1]
    ts0_o[...] = t0 * disv
    ts1_o[...] = t1 * disv
    tf0_o[...] = t0 * invv
    tf1_o[...] = t1 * invv


def _tc_mid(a00, a01, a10, a11, x0, x1, dis, inv, W1, b1, W2):
    sds = jax.ShapeDtypeStruct((NR, 128), jnp.float32)
    vspec = pl.BlockSpec(memory_space=pltpu.VMEM)
    sspec = pl.BlockSpec(memory_space=pltpu.SMEM)
    return pl.pallas_call(
        _tc_mid_body,
        out_shape=[sds, sds, sds, sds],
        in_specs=[vspec] * 8 + [sspec, sspec, sspec],
        out_specs=[vspec] * 4,
    )(a00, a01, a10, a11, x0, x1, dis, inv, W1, b1, W2)


def _tc_final_body(b00, b01, b10, b11, dis, tf0, tf1, batch_r, b2, out):
    disv = dis[...]
    y0 = disv * (b00[...] + b10[...]) + tf0[...] + b2[0]
    y1 = disv * (b01[...] + b11[...]) + tf1[...] + b2[1]
    node = (lax.broadcasted_iota(jnp.int32, (NR, 128), 0) * 128
            + lax.broadcasted_iota(jnp.int32, (NR, 128), 1))
    valid = node < N_NODES
    neg = jnp.float32(-jnp.inf)
    bt = batch_r[...]
    p0 = []
    p1 = []
    for g in range(G_SEG):
        m = jnp.logical_and(bt == g, valid)
        p0.append(jnp.max(jnp.where(m, y0, neg)))
        p1.append(jnp.max(jnp.where(m, y1, neg)))
    pa = jnp.stack(p0)
    pb = jnp.stack(p1)
    mx = jnp.maximum(pa, pb)
    lse = mx + jnp.log(jnp.exp(pa - mx) + jnp.exp(pb - mx))
    out[0, :] = pa - lse
    out[1, :] = pb - lse


def _tc_final(b00, b01, b10, b11, dis, tf0, tf1, batch_r, b2):
    vspec = pl.BlockSpec(memory_space=pltpu.VMEM)
    sspec = pl.BlockSpec(memory_space=pltpu.SMEM)
    return pl.pallas_call(
        _tc_final_body,
        out_shape=jax.ShapeDtypeStruct((2, G_SEG), jnp.float32),
        in_specs=[vspec] * 8 + [sspec],
        out_specs=vspec,
    )(b00, b01, b10, b11, dis, tf0, tf1, batch_r, b2)


def _soa(v):
    return jnp.pad(v, (0, N_PAD - v.shape[0])).reshape(NR, 128)


def kernel(x, ei, batch, W1, b1, W2, b2):
    E = ei.shape[1]
    rpw = -(-E // (NW * CHUNK * INNER)) * INNER
    rows = rpw * NW
    e_pad = rows * CHUNK

    src = jnp.concatenate(
        [ei[0], jnp.full((e_pad - E,), N_NODES, jnp.int32)]).reshape(rows, CHUNK)
    dst = jnp.concatenate(
        [ei[1], jnp.full((e_pad - E,), N_NODES, jnp.int32)]).reshape(rows, CHUNK)

    x0 = _soa(x[:, 0])
    x1 = _soa(x[:, 1])
    zeros2 = jnp.zeros((N_PAD, 2), jnp.float32)

    degp = _sc_deg(dst)
    d0 = degp[0].reshape(NR, 128)
    d1 = degp[1].reshape(NR, 128)

    dis, inv, xs0, xs1 = _tc_prep(d0, d1, x0, x1)

    xs = jnp.stack([xs0.reshape(-1), xs1.reshape(-1)], axis=1)
    acc1 = _sc_spmv(src, dst, xs, zeros2)

    a00 = acc1[0, :, 0].reshape(NR, 128)
    a01 = acc1[0, :, 1].reshape(NR, 128)
    a10 = acc1[1, :, 0].reshape(NR, 128)
    a11 = acc1[1, :, 1].reshape(NR, 128)

    ts0, ts1, tf0, tf1 = _tc_mid(a00, a01, a10, a11, x0, x1, dis, inv,
                                 W1, b1, W2)

    ts = jnp.stack([ts0.reshape(-1), ts1.reshape(-1)], axis=1)
    acc2 = _sc_spmv(src, dst, ts, zeros2)

    b00 = acc2[0, :, 0].reshape(NR, 128)
    b01 = acc2[0, :, 1].reshape(NR, 128)
    b10 = acc2[1, :, 0].reshape(NR, 128)
    b11 = acc2[1, :, 1].reshape(NR, 128)

    batch_r = jnp.pad(batch, (0, N_PAD - batch.shape[0]),
                      constant_values=G_SEG - 1).reshape(NR, 128)

    out = _tc_final(b00, b01, b10, b11, dis, tf0, tf1, batch_r, b2)
    return out.T


# speedup vs baseline: 56.0475x; 56.0475x over previous
import functools

import jax
import jax.numpy as jnp
from jax import lax
from jax.experimental import pallas as pl
from jax.experimental.pallas import tpu as pltpu
from jax.experimental.pallas import tpu_sc as plsc

N_NODES = 100000
G_SEG = 64
N_PAD = 100352
NR = 784
NPW = N_PAD // 16
NCORE = 2
NSUB = 16
NW = NCORE * NSUB
CHUNK = 128
INNER = 8


def _mesh():
    return plsc.VectorSubcoreMesh(
        core_axis_name="c", subcore_axis_name="s",
        num_cores=NCORE, num_subcores=NSUB)


def _zero_vmem_1d(ref, n):
    def body(i, _):
        ref[pl.ds(i * 16, 16)] = jnp.zeros((16,), jnp.float32)
        return 0
    lax.fori_loop(0, n // 16, body, 0)


def _sc_deg_body(rows_per_worker, dst_hbm, out_hbm, dstbuf, ones_v, zbuf,
                 obuf, deg_sp, gsem):
    c = lax.axis_index("c")
    s = lax.axis_index("s")
    w = c * NSUB + s

    _zero_vmem_1d(zbuf, NPW)

    def fill_ones(i, _):
        ones_v[pl.ds(i * 16, 16)] = jnp.ones((16,), jnp.float32)
        return 0
    lax.fori_loop(0, CHUNK // 16, fill_ones, 0)

    pltpu.sync_copy(zbuf, deg_sp.at[pl.ds(s * NPW, NPW)])
    plsc.subcore_barrier()

    n_outer = rows_per_worker // INNER
    row0 = w * rows_per_worker

    def outer(k, _):
        r0 = row0 + k * INNER
        pltpu.sync_copy(dst_hbm.at[pl.ds(r0, INNER)], dstbuf)
        for j in range(INNER):
            pltpu.sync_copy(ones_v, deg_sp.at[dstbuf.at[j]], add=True)
        return 0
    lax.fori_loop(0, n_outer, outer, 0)

    plsc.subcore_barrier()
    pltpu.sync_copy(deg_sp.at[pl.ds(s * NPW, NPW)], obuf)
    pltpu.sync_copy(obuf, out_hbm.at[c, pl.ds(s * NPW, NPW)])


def _sc_deg(dst2d):
    rows = dst2d.shape[0]
    rpw = rows // NW
    body = functools.partial(_sc_deg_body, rpw)
    return pl.kernel(
        body,
        out_type=jax.ShapeDtypeStruct((NCORE, N_PAD), jnp.float32),
        mesh=_mesh(),
        scratch_types=[
            pltpu.VMEM((INNER, CHUNK), jnp.int32),
            pltpu.VMEM((CHUNK,), jnp.float32),
            pltpu.VMEM((NPW,), jnp.float32),
            pltpu.VMEM((NPW,), jnp.float32),
            pltpu.VMEM_SHARED((N_PAD,), jnp.float32),
            pltpu.SemaphoreType.DMA,
        ],
        compiler_params=pltpu.CompilerParams(use_tc_tiling_on_sc=False),
    )(dst2d)


def _sc_spmv_body(rows_per_worker, src_hbm, dst_hbm, v0_hbm, v1_hbm,
                  out_hbm, srcbuf, dstbuf, g0buf, g1buf, zbuf,
                  y0_sp, y1_sp, gsem):
    c = lax.axis_index("c")
    s = lax.axis_index("s")
    w = c * NSUB + s

    _zero_vmem_1d(zbuf, NPW)
    pltpu.sync_copy(zbuf, y0_sp.at[pl.ds(s * NPW, NPW)])
    pltpu.sync_copy(zbuf, y1_sp.at[pl.ds(s * NPW, NPW)])
    plsc.subcore_barrier()

    n_outer = rows_per_worker // INNER
    row0 = w * rows_per_worker

    def outer(k, _):
        r0 = row0 + k * INNER
        pltpu.sync_copy(src_hbm.at[pl.ds(r0, INNER)], srcbuf)
        pltpu.sync_copy(dst_hbm.at[pl.ds(r0, INNER)], dstbuf)
        descs = []
        for j in range(INNER):
            descs.append(pltpu.async_copy(
                v0_hbm.at[srcbuf.at[j]], g0buf.at[j], gsem))
            descs.append(pltpu.async_copy(
                v1_hbm.at[srcbuf.at[j]], g1buf.at[j], gsem))
        for d in descs:
            d.wait()
        for j in range(INNER):
            pltpu.sync_copy(g0buf.at[j], y0_sp.at[dstbuf.at[j]], add=True)
            pltpu.sync_copy(g1buf.at[j], y1_sp.at[dstbuf.at[j]], add=True)
        return 0
    lax.fori_loop(0, n_outer, outer, 0)

    plsc.subcore_barrier()
    pltpu.sync_copy(y0_sp.at[pl.ds(s * NPW, NPW)], zbuf)
    pltpu.sync_copy(zbuf, out_hbm.at[c, 0, pl.ds(s * NPW, NPW)])
    pltpu.sync_copy(y1_sp.at[pl.ds(s * NPW, NPW)], zbuf)
    pltpu.sync_copy(zbuf, out_hbm.at[c, 1, pl.ds(s * NPW, NPW)])


def _sc_spmv(src2d, dst2d, v0, v1):
    rows = src2d.shape[0]
    rpw = rows // NW
    body = functools.partial(_sc_spmv_body, rpw)
    return pl.kernel(
        body,
        out_type=jax.ShapeDtypeStruct((NCORE, 2, N_PAD), jnp.float32),
        mesh=_mesh(),
        scratch_types=[
            pltpu.VMEM((INNER, CHUNK), jnp.int32),
            pltpu.VMEM((INNER, CHUNK), jnp.int32),
            pltpu.VMEM((INNER, CHUNK), jnp.float32),
            pltpu.VMEM((INNER, CHUNK), jnp.float32),
            pltpu.VMEM((NPW,), jnp.float32),
            pltpu.VMEM_SHARED((N_PAD,), jnp.float32),
            pltpu.VMEM_SHARED((N_PAD,), jnp.float32),
            pltpu.SemaphoreType.DMA,
        ],
        compiler_params=pltpu.CompilerParams(use_tc_tiling_on_sc=False),
    )(src2d, dst2d, v0, v1)


def _tc_prep_body(d0, d1, x0, x1, dis_o, inv_o, xs0_o, xs1_o):
    deg = d0[...] + d1[...] + 1.0
    dis = lax.rsqrt(deg)
    inv = 1.0 / deg
    dis_o[...] = dis
    inv_o[...] = inv
    xs0_o[...] = x0[...] * dis
    xs1_o[...] = x1[...] * dis


def _tc_prep(d0, d1, x0, x1):
    sds = jax.ShapeDtypeStruct((NR, 128), jnp.float32)
    return pl.pallas_call(
        _tc_prep_body,
        out_shape=[sds, sds, sds, sds],
    )(d0, d1, x0, x1)


def _tc_mid_body(a00, a01, a10, a11, x0, x1, dis, inv, W1, b1, W2,
                 ts0_o, ts1_o, tf0_o, tf1_o):
    disv = dis[...]
    invv = inv[...]
    y0 = disv * (a00[...] + a10[...]) + x0[...] * invv
    y1 = disv * (a01[...] + a11[...]) + x1[...] * invv
    t0 = jnp.zeros_like(y0)
    t1 = jnp.zeros_like(y0)
    for j in range(64):
        h = jnp.maximum(y0 * W1[0, j] + y1 * W1[1, j] + b1[j], 0.0)
        t0 = t0 + h * W2[j, 0]
        t1 = t1 + h * W2[j, 1]
    ts0_o[...] = t0 * disv
    ts1_o[...] = t1 * disv
    tf0_o[...] = t0 * invv
    tf1_o[...] = t1 * invv


def _tc_mid(a00, a01, a10, a11, x0, x1, dis, inv, W1, b1, W2):
    sds = jax.ShapeDtypeStruct((NR, 128), jnp.float32)
    vspec = pl.BlockSpec(memory_space=pltpu.VMEM)
    sspec = pl.BlockSpec(memory_space=pltpu.SMEM)
    return pl.pallas_call(
        _tc_mid_body,
        out_shape=[sds, sds, sds, sds],
        in_specs=[vspec] * 8 + [sspec, sspec, sspec],
        out_specs=[vspec] * 4,
    )(a00, a01, a10, a11, x0, x1, dis, inv, W1, b1, W2)


def _tc_final_body(b00, b01, b10, b11, dis, tf0, tf1, batch_r, b2, out):
    disv = dis[...]
    y0 = disv * (b00[...] + b10[...]) + tf0[...] + b2[0]
    y1 = disv * (b01[...] + b11[...]) + tf1[...] + b2[1]
    node = (lax.broadcasted_iota(jnp.int32, (NR, 128), 0) * 128
            + lax.broadcasted_iota(jnp.int32, (NR, 128), 1))
    valid = node < N_NODES
    neg = jnp.float32(-jnp.inf)
    bt = batch_r[...]
    p0 = []
    p1 = []
    for g in range(G_SEG):
        m = jnp.logical_and(bt == g, valid)
        p0.append(jnp.max(jnp.where(m, y0, neg)))
        p1.append(jnp.max(jnp.where(m, y1, neg)))
    pa = jnp.stack(p0)
    pb = jnp.stack(p1)
    mx = jnp.maximum(pa, pb)
    lse = mx + jnp.log(jnp.exp(pa - mx) + jnp.exp(pb - mx))
    out[0, :] = pa - lse
    out[1, :] = pb - lse


def _tc_final(b00, b01, b10, b11, dis, tf0, tf1, batch_r, b2):
    vspec = pl.BlockSpec(memory_space=pltpu.VMEM)
    sspec = pl.BlockSpec(memory_space=pltpu.SMEM)
    return pl.pallas_call(
        _tc_final_body,
        out_shape=jax.ShapeDtypeStruct((2, G_SEG), jnp.float32),
        in_specs=[vspec] * 8 + [sspec],
        out_specs=vspec,
    )(b00, b01, b10, b11, dis, tf0, tf1, batch_r, b2)


def _soa(v):
    return jnp.pad(v, (0, N_PAD - v.shape[0])).reshape(NR, 128)


def kernel(x, ei, batch, W1, b1, W2, b2):
    E = ei.shape[1]
    rpw = -(-E // (NW * CHUNK * INNER)) * INNER
    rows = rpw * NW
    e_pad = rows * CHUNK

    src = jnp.concatenate(
        [ei[0], jnp.full((e_pad - E,), N_NODES, jnp.int32)]).reshape(rows, CHUNK)
    dst = jnp.concatenate(
        [ei[1], jnp.full((e_pad - E,), N_NODES, jnp.int32)]).reshape(rows, CHUNK)

    x0 = _soa(x[:, 0])
    x1 = _soa(x[:, 1])

    degp = _sc_deg(dst)
    d0 = degp[0].reshape(NR, 128)
    d1 = degp[1].reshape(NR, 128)

    dis, inv, xs0, xs1 = _tc_prep(d0, d1, x0, x1)

    acc1 = _sc_spmv(src, dst, xs0.reshape(-1), xs1.reshape(-1))

    a00 = acc1[0, 0].reshape(NR, 128)
    a01 = acc1[0, 1].reshape(NR, 128)
    a10 = acc1[1, 0].reshape(NR, 128)
    a11 = acc1[1, 1].reshape(NR, 128)

    ts0, ts1, tf0, tf1 = _tc_mid(a00, a01, a10, a11, x0, x1, dis, inv,
                                 W1, b1, W2)

    acc2 = _sc_spmv(src, dst, ts0.reshape(-1), ts1.reshape(-1))

    b00 = acc2[0, 0].reshape(NR, 128)
    b01 = acc2[0, 1].reshape(NR, 128)
    b10 = acc2[1, 0].reshape(NR, 128)
    b11 = acc2[1, 1].reshape(NR, 128)

    batch_r = jnp.pad(batch, (0, N_PAD - batch.shape[0]),
                      constant_values=G_SEG - 1).reshape(NR, 128)

    out = _tc_final(b00, b01, b10, b11, dis, tf0, tf1, batch_r, b2)
    return out.T
